# Initial kernel scaffold; baseline (speedup 1.0000x reference)
import jax
import jax.numpy as jnp
from jax import lax
from jax.experimental import pallas as pl
from jax.experimental.pallas import tpu as pltpu
from jax.experimental.pallas import tpu_sc as plsc

N = 10000
D = 128
E = 320000

DP = 144
NC = 2
NS = 16
CHUNK = 128
CPT = 79
EPT = CHUNK * CPT
E_PAD = EPT * NC * NS
RPT = 632
NPAD = RPT * NS


def _sc_body(xa, srcp, dstp, zrow, out, src_v, dst_v, rows_v, agg_sh, sem):
    c = lax.axis_index("c")
    s = lax.axis_index("s")
    r0 = s * RPT
    pltpu.sync_copy(zrow, agg_sh.at[pl.ds(r0, RPT)])
    plsc.subcore_barrier()
    base = (c * NS + s) * EPT

    def chunk(i, carry):
        off = base + i * CHUNK
        pltpu.sync_copy(srcp.at[pl.ds(off, CHUNK)], src_v)
        pltpu.sync_copy(dstp.at[pl.ds(off, CHUNK)], dst_v)
        pltpu.async_copy(xa.at[src_v], rows_v, sem).wait()
        pltpu.sync_copy(rows_v, agg_sh.at[dst_v], add=True)
        return carry

    lax.fori_loop(0, CPT, chunk, 0)
    plsc.subcore_barrier()
    pltpu.sync_copy(agg_sh.at[pl.ds(r0, RPT)],
                    out.at[pl.ds(c * NPAD + r0, RPT)])


_sc_call = pl.kernel(
    _sc_body,
    out_type=jax.ShapeDtypeStruct((NC * NPAD, DP), jnp.float32),
    mesh=plsc.VectorSubcoreMesh(core_axis_name="c", subcore_axis_name="s"),
    scratch_types=[
        pltpu.VMEM((CHUNK,), jnp.int32),
        pltpu.VMEM((CHUNK,), jnp.int32),
        pltpu.VMEM((CHUNK, DP), jnp.float32),
        pltpu.VMEM_SHARED((NPAD, DP), jnp.float32),
        pltpu.SemaphoreType.DMA,
    ],
)


BN = 2000


def _tc_body(p_ref, wa_ref, out_ref):
    p = p_ref[0] + p_ref[1]
    num = jnp.dot(p, wa_ref[...], preferred_element_type=jnp.float32)
    cnt = p[:, D:D + 1]
    out_ref[...] = jnp.maximum(num / jnp.maximum(cnt, 1.0), 0.0)


def kernel(x, edge_index, W, b):
    src = edge_index[0]
    dst = edge_index[1]
    pad = E_PAD - E
    srcp = jnp.concatenate([src, jnp.full((pad,), N, jnp.int32)])
    dstp = jnp.concatenate([dst, jnp.full((pad,), N, jnp.int32)])
    xa = jnp.zeros((NPAD, DP), jnp.float32)
    xa = xa.at[:N, :D].set(x).at[:N, D].set(1.0)
    zrow = jnp.zeros((RPT, DP), jnp.float32)

    partials = _sc_call(xa, srcp, dstp, zrow).reshape(NC, NPAD, DP)

    wa = jnp.zeros((DP, D), jnp.float32).at[:D].set(W.T).at[D].set(b)
    out = pl.pallas_call(
        _tc_body,
        grid=(N // BN,),
        in_specs=[
            pl.BlockSpec((NC, BN, DP), lambda i: (0, i, 0)),
            pl.BlockSpec((DP, D), lambda i: (0, 0)),
        ],
        out_specs=pl.BlockSpec((BN, D), lambda i: (i, 0)),
        out_shape=jax.ShapeDtypeStruct((N, D), jnp.float32),
    )(partials, wa)
    return out


# speedup vs baseline: 3.5950x; 3.5950x over previous
import jax
import jax.numpy as jnp
from jax import lax
from jax.experimental import pallas as pl
from jax.experimental.pallas import tpu as pltpu
from jax.experimental.pallas import tpu_sc as plsc

N = 10000
D = 128
E = 320000

DP = 144
NC = 2
NS = 16
CHUNK = 128
CPT = 79
EPT = CHUNK * CPT
E_PAD = EPT * NC * NS
RPT = 632
NPAD = RPT * NS


def _sc_body(xa, srcp, dstp, zrow, out, src_v, dst_v, rows_v, agg_sh, sem):
    c = lax.axis_index("c")
    s = lax.axis_index("s")
    r0 = s * RPT
    pltpu.sync_copy(zrow, agg_sh.at[pl.ds(r0, RPT)])
    plsc.subcore_barrier()
    base = (c * NS + s) * EPT

    def chunk(i, carry):
        off = base + i * CHUNK
        pltpu.sync_copy(srcp.at[pl.ds(off, CHUNK)], src_v)
        pltpu.sync_copy(dstp.at[pl.ds(off, CHUNK)], dst_v)
        pltpu.async_copy(xa.at[src_v], rows_v, sem).wait()
        pltpu.sync_copy(rows_v, agg_sh.at[dst_v], add=True)
        return carry

    lax.fori_loop(0, CPT, chunk, 0)
    plsc.subcore_barrier()
    pltpu.sync_copy(agg_sh.at[pl.ds(r0, RPT)],
                    out.at[pl.ds(c * NPAD + r0, RPT)])


_sc_call = pl.kernel(
    _sc_body,
    out_type=jax.ShapeDtypeStruct((NC * NPAD, DP), jnp.float32),
    mesh=plsc.VectorSubcoreMesh(core_axis_name="c", subcore_axis_name="s"),
    scratch_types=[
        pltpu.VMEM((CHUNK,), jnp.int32),
        pltpu.VMEM((CHUNK,), jnp.int32),
        pltpu.VMEM((CHUNK, DP), jnp.float32),
        pltpu.VMEM_SHARED((NPAD, DP), jnp.float32),
        pltpu.SemaphoreType.DMA,
    ],
    compiler_params=pltpu.CompilerParams(use_tc_tiling_on_sc=False),
)


BN = 2000


def _tc_body(p_ref, wa_ref, out_ref):
    p = p_ref[0] + p_ref[1]
    num = jnp.dot(p, wa_ref[...], preferred_element_type=jnp.float32)
    cnt = p[:, D:D + 1]
    out_ref[...] = jnp.maximum(num / jnp.maximum(cnt, 1.0), 0.0)


def kernel(x, edge_index, W, b):
    src = edge_index[0]
    dst = edge_index[1]
    pad = E_PAD - E
    srcp = jnp.concatenate([src, jnp.full((pad,), N, jnp.int32)])
    dstp = jnp.concatenate([dst, jnp.full((pad,), N, jnp.int32)])
    xa = jnp.zeros((NPAD, DP), jnp.float32)
    xa = xa.at[:N, :D].set(x).at[:N, D].set(1.0)
    zrow = jnp.zeros((RPT, DP), jnp.float32)

    partials = _sc_call(xa, srcp, dstp, zrow).reshape(NC, NPAD, DP)

    wa = jnp.zeros((DP, D), jnp.float32).at[:D].set(W.T).at[D].set(b)
    out = pl.pallas_call(
        _tc_body,
        grid=(N // BN,),
        in_specs=[
            pl.BlockSpec((NC, BN, DP), lambda i: (0, i, 0)),
            pl.BlockSpec((DP, D), lambda i: (0, 0)),
        ],
        out_specs=pl.BlockSpec((BN, D), lambda i: (i, 0)),
        out_shape=jax.ShapeDtypeStruct((N, D), jnp.float32),
    )(partials, wa)
    return out


# speedup vs baseline: 4.5057x; 1.2533x over previous
import jax
import jax.numpy as jnp
from jax import lax
from jax.experimental import pallas as pl
from jax.experimental.pallas import tpu as pltpu
from jax.experimental.pallas import tpu_sc as plsc

N = 10000
D = 128
E = 320000

DP = 144
NC = 2
NS = 16
CHUNK = 128
CPT = 79
EPT = CHUNK * CPT
E_PAD = EPT * NC * NS
RPT = 632
NPAD = RPT * NS

NBUF = 2


def _sc_body(xa, srcp, dstp, zrow, out, sidx, didx, rows, agg_sh, sems):
    c = lax.axis_index("c")
    s = lax.axis_index("s")
    r0 = s * RPT
    pltpu.sync_copy(zrow, agg_sh.at[pl.ds(r0, RPT)])
    base = (c * NS + s) * EPT
    pltpu.sync_copy(srcp.at[pl.ds(base, CHUNK)], sidx.at[0])
    pltpu.sync_copy(dstp.at[pl.ds(base, CHUNK)], didx.at[0])
    plsc.subcore_barrier()
    pltpu.async_copy(xa.at[sidx.at[0]], rows.at[0], sems.at[0])

    def chunk(i, carry):
        buf = lax.rem(i, NBUF)
        nxt = lax.rem(i + 1, NBUF)

        @pl.when(i + 1 < CPT)
        def _():
            off = base + (i + 1) * CHUNK
            pltpu.sync_copy(srcp.at[pl.ds(off, CHUNK)], sidx.at[nxt])
            pltpu.sync_copy(dstp.at[pl.ds(off, CHUNK)], didx.at[nxt])
            pltpu.async_copy(xa.at[sidx.at[nxt]], rows.at[nxt],
                             sems.at[nxt])

        pltpu.make_async_copy(xa.at[sidx.at[buf]], rows.at[buf],
                              sems.at[buf]).wait()
        pltpu.sync_copy(rows.at[buf], agg_sh.at[didx.at[buf]], add=True)
        return carry

    lax.fori_loop(0, CPT, chunk, 0)
    plsc.subcore_barrier()
    pltpu.sync_copy(agg_sh.at[pl.ds(r0, RPT)],
                    out.at[pl.ds(c * NPAD + r0, RPT)])


_sc_call = pl.kernel(
    _sc_body,
    out_type=jax.ShapeDtypeStruct((NC * NPAD, DP), jnp.float32),
    mesh=plsc.VectorSubcoreMesh(core_axis_name="c", subcore_axis_name="s"),
    scratch_types=[
        pltpu.VMEM((NBUF, CHUNK), jnp.int32),
        pltpu.VMEM((NBUF, CHUNK), jnp.int32),
        pltpu.VMEM((NBUF, CHUNK, DP), jnp.float32),
        pltpu.VMEM_SHARED((NPAD, DP), jnp.float32),
        pltpu.SemaphoreType.DMA((NBUF,)),
    ],
    compiler_params=pltpu.CompilerParams(use_tc_tiling_on_sc=False),
)


BN = 2000


def _tc_body(p_ref, wa_ref, out_ref):
    p = p_ref[0] + p_ref[1]
    num = jnp.dot(p, wa_ref[...], preferred_element_type=jnp.float32)
    cnt = p[:, D:D + 1]
    out_ref[...] = jnp.maximum(num / jnp.maximum(cnt, 1.0), 0.0)


def kernel(x, edge_index, W, b):
    src = edge_index[0]
    dst = edge_index[1]
    pad = E_PAD - E
    srcp = jnp.concatenate([src, jnp.full((pad,), N, jnp.int32)])
    dstp = jnp.concatenate([dst, jnp.full((pad,), N, jnp.int32)])
    xa = jnp.zeros((NPAD, DP), jnp.float32)
    xa = xa.at[:N, :D].set(x).at[:N, D].set(1.0)
    zrow = jnp.zeros((RPT, DP), jnp.float32)

    partials = _sc_call(xa, srcp, dstp, zrow).reshape(NC, NPAD, DP)

    wa = jnp.zeros((DP, D), jnp.float32).at[:D].set(W.T).at[D].set(b)
    out = pl.pallas_call(
        _tc_body,
        grid=(N // BN,),
        in_specs=[
            pl.BlockSpec((NC, BN, DP), lambda i: (0, i, 0)),
            pl.BlockSpec((DP, D), lambda i: (0, 0)),
        ],
        out_specs=pl.BlockSpec((BN, D), lambda i: (i, 0)),
        out_shape=jax.ShapeDtypeStruct((N, D), jnp.float32),
    )(partials, wa)
    return out


# speedup vs baseline: 5.7605x; 1.2785x over previous
import jax
import jax.numpy as jnp
from jax import lax
from jax.experimental import pallas as pl
from jax.experimental.pallas import tpu as pltpu
from jax.experimental.pallas import tpu_sc as plsc

N = 10000
D = 128
E = 320000

DP = 144
NC = 2
NS = 16
CHUNK = 112
G = 9
NG = 10
GE = G * CHUNK
CPT = G * NG
EPT = CHUNK * CPT
E_PAD = EPT * NC * NS
RPT = 626
NPAD = RPT * NS

NBUF = 2


def _sc_body(xa, srcp, dstp, zrow, out, sidx, didx, dstc, rows, agg_sh,
             gsem, ssem, isem):
    c = lax.axis_index("c")
    s = lax.axis_index("s")
    r0 = s * RPT
    pltpu.sync_copy(zrow, agg_sh.at[pl.ds(r0, RPT)])
    base = (c * NS + s) * EPT
    pltpu.sync_copy(srcp.at[pl.ds(base, GE)], sidx.at[0])
    pltpu.sync_copy(dstp.at[pl.ds(base, GE)], didx.at[0])
    plsc.subcore_barrier()
    pltpu.async_copy(xa.at[sidx.at[0, pl.ds(0, CHUNK)]], rows.at[0],
                     gsem.at[0])
    pltpu.async_copy(srcp.at[pl.ds(base + GE, GE)], sidx.at[1], isem)
    pltpu.async_copy(dstp.at[pl.ds(base + GE, GE)], didx.at[1], isem)

    def group(g, carry):
        p = lax.rem(g, 2)
        q = lax.rem(g + 1, 2)
        for j in range(G):
            i = g * G + j
            buf = lax.rem(i, NBUF)
            nxt = lax.rem(i + 1, NBUF)

            if j + 1 < G:
                @pl.when(i >= 1)
                def _():
                    pltpu.make_async_copy(rows.at[nxt],
                                          agg_sh.at[dstc.at[nxt]],
                                          ssem.at[nxt]).wait()

                pltpu.async_copy(
                    xa.at[sidx.at[p, pl.ds((j + 1) * CHUNK, CHUNK)]],
                    rows.at[nxt], gsem.at[nxt])
            else:
                @pl.when(g + 1 < NG)
                def _():
                    pltpu.make_async_copy(rows.at[nxt],
                                          agg_sh.at[dstc.at[nxt]],
                                          ssem.at[nxt]).wait()
                    pltpu.make_async_copy(srcp.at[pl.ds(base, GE)],
                                          sidx.at[q], isem).wait()
                    pltpu.make_async_copy(dstp.at[pl.ds(base, GE)],
                                          didx.at[q], isem).wait()
                    pltpu.async_copy(xa.at[sidx.at[q, pl.ds(0, CHUNK)]],
                                     rows.at[nxt], gsem.at[nxt])

            for k in range(CHUNK // 16):
                dstc[buf, pl.ds(k * 16, 16)] = (
                    didx[p, pl.ds(j * CHUNK + k * 16, 16)])

            pltpu.make_async_copy(
                xa.at[sidx.at[p, pl.ds(j * CHUNK, CHUNK)]],
                rows.at[buf], gsem.at[buf]).wait()
            pltpu.async_copy(rows.at[buf], agg_sh.at[dstc.at[buf]],
                             ssem.at[buf], add=True)

        @pl.when(g + 2 < NG)
        def _():
            off = base + (g + 2) * GE
            pltpu.async_copy(srcp.at[pl.ds(off, GE)], sidx.at[p], isem)
            pltpu.async_copy(dstp.at[pl.ds(off, GE)], didx.at[p], isem)

        return carry

    lax.fori_loop(0, NG, group, 0)
    for b in range(NBUF):
        pltpu.make_async_copy(rows.at[b], agg_sh.at[dstc.at[b]],
                              ssem.at[b]).wait()
    plsc.subcore_barrier()
    pltpu.sync_copy(agg_sh.at[pl.ds(r0, RPT)],
                    out.at[pl.ds(c * NPAD + r0, RPT)])


_sc_call = pl.kernel(
    _sc_body,
    out_type=jax.ShapeDtypeStruct((NC * NPAD, DP), jnp.float32),
    mesh=plsc.VectorSubcoreMesh(core_axis_name="c", subcore_axis_name="s"),
    scratch_types=[
        pltpu.VMEM((2, GE), jnp.int32),
        pltpu.VMEM((2, GE), jnp.int32),
        pltpu.VMEM((NBUF, CHUNK), jnp.int32),
        pltpu.VMEM((NBUF, CHUNK, DP), jnp.float32),
        pltpu.VMEM_SHARED((NPAD, DP), jnp.float32),
        pltpu.SemaphoreType.DMA((NBUF,)),
        pltpu.SemaphoreType.DMA((NBUF,)),
        pltpu.SemaphoreType.DMA,
    ],
    compiler_params=pltpu.CompilerParams(use_tc_tiling_on_sc=False),
)


BN = 2000


def _tc_body(p_ref, wa_ref, out_ref):
    p = p_ref[0] + p_ref[1]
    num = jnp.dot(p, wa_ref[...], preferred_element_type=jnp.float32)
    cnt = p[:, D:D + 1]
    out_ref[...] = jnp.maximum(num / jnp.maximum(cnt, 1.0), 0.0)


def kernel(x, edge_index, W, b):
    src = edge_index[0]
    dst = edge_index[1]
    pad = E_PAD - E
    srcp = jnp.concatenate([src, jnp.full((pad,), N, jnp.int32)])
    dstp = jnp.concatenate([dst, jnp.full((pad,), N, jnp.int32)])
    xa = jnp.zeros((NPAD, DP), jnp.float32)
    xa = xa.at[:N, :D].set(x).at[:N, D].set(1.0)
    zrow = jnp.zeros((RPT, DP), jnp.float32)

    partials = _sc_call(xa, srcp, dstp, zrow).reshape(NC, NPAD, DP)

    wa = jnp.zeros((DP, D), jnp.float32).at[:D].set(W.T).at[D].set(b)
    out = pl.pallas_call(
        _tc_body,
        grid=(N // BN,),
        in_specs=[
            pl.BlockSpec((NC, BN, DP), lambda i: (0, i, 0)),
            pl.BlockSpec((DP, D), lambda i: (0, 0)),
        ],
        out_specs=pl.BlockSpec((BN, D), lambda i: (i, 0)),
        out_shape=jax.ShapeDtypeStruct((N, D), jnp.float32),
    )(partials, wa)
    return out
